# baseline (speedup 1.0000x reference)
import functools

import jax
import jax.numpy as jnp
from jax import lax
from jax.experimental import pallas as pl
from jax.experimental.pallas import tpu as pltpu
from jax.experimental.pallas import tpu_sc as plsc

VOC = 1000000
NB_FULL = 7812
NSB = 2604
SBW = 82
R_ROWS = 500000

NC, NS = 2, 16
NW = NC * NS

_mesh = plsc.VectorSubcoreMesh(
    core_axis_name="c", subcore_axis_name="s", num_cores=NC, num_subcores=NS)

_params = pltpu.CompilerParams(
    use_tc_tiling_on_sc=True, needs_layout_passes=False)


def _splat(x):
  return jnp.zeros((16,), jnp.int32) + x


@functools.partial(
    pl.kernel,
    out_type=jax.ShapeDtypeStruct((R_ROWS, 128), jnp.float32),
    mesh=_mesh,
    compiler_params=_params,
    scratch_types=[
        pltpu.VMEM((2, 64, 384), jnp.float32),
        pltpu.VMEM((2, 192, 128), jnp.float32),
        pltpu.VMEM((64, 64), jnp.float32),
        pltpu.VMEM((32, 128), jnp.float32),
        pltpu.SemaphoreType.DMA,
        pltpu.SemaphoreType.DMA,
        pltpu.SemaphoreType.DMA,
        pltpu.SemaphoreType.DMA,
    ],
)
def _k1(tableT, r_out, in_v, out_v, tin_v, tout_v, is0, is1, os0, os1):
  w = lax.axis_index("s") * NC + lax.axis_index("c")
  iota = lax.iota(jnp.int32, 16)
  isems = (is0, is1)
  osems = (os0, os1)
  j0 = w * SBW

  def start_in(j, b):
    pltpu.async_copy(tableT.at[:, pl.ds(j * 384, 384)], in_v.at[b], isems[b])

  def wait_in(b):
    pltpu.make_async_copy(
        tableT.at[:, pl.ds(0, 384)], in_v.at[b], isems[b]).wait()

  def start_out(j, b):
    pltpu.async_copy(out_v.at[b], r_out.at[pl.ds(j * 192, 192)], osems[b])

  def wait_out(b):
    pltpu.make_async_copy(
        out_v.at[b], r_out.at[pl.ds(0, 192)], osems[b]).wait()

  def transpose_rows(src, dst, nrows):
    @plsc.parallel_loop(0, nrows, unroll=4)
    def _(r):
      for half in range(2):
        lvec = _splat(2 * r + half)
        for c0 in range(0, 64, 16):
          v = plsc.load_gather(src, [iota + c0, lvec])
          dst[r, pl.ds(half * 64 + c0, 16)] = v

  @pl.when(j0 < NSB)
  def _():
    start_in(j0, 0)

  @pl.when(j0 + 1 < NSB)
  def _():
    start_in(j0 + 1, 1)

  @pl.loop(0, SBW + 1, step=2)
  def _(k):
    for half in range(2):
      kk = k + half
      j = j0 + kk
      b = half

      @pl.when((j < NSB) & (kk < SBW))
      def _():
        wait_in(b)

        @pl.when(kk >= 2)
        def _():
          wait_out(b)

        transpose_rows(in_v.at[b], out_v.at[b], 192)
        start_out(j, b)

        @pl.when((j + 2 < NSB) & (kk + 2 < SBW))
        def _():
          start_in(j + 2, b)

  @pl.when(j0 < NSB)
  def _():
    wait_out(0)

  @pl.when(j0 + 1 < NSB)
  def _():
    wait_out(1)

  @pl.when(w == 0)
  def _():
    pltpu.sync_copy(tableT.at[:, pl.ds(NB_FULL * 128, 64)], tin_v)

    @plsc.parallel_loop(0, 32, unroll=4)
    def _(r):
      for half in range(2):
        lvec = _splat(2 * r + half)
        for c0 in range(0, 64, 16):
          v = plsc.load_gather(tin_v, [iota + c0, lvec])
          tout_v[r, pl.ds(half * 64 + c0, 16)] = v

    pltpu.sync_copy(tout_v, r_out.at[pl.ds(NB_FULL * 64, 32)])


@functools.partial(
    pl.kernel,
    out_type=jax.ShapeDtypeStruct((200, 64, 4096), jnp.float32),
    mesh=_mesh,
    compiler_params=_params,
    scratch_types=[
        pltpu.VMEM((200, 128), jnp.int32),
        pltpu.VMEM((200, 128), jnp.int32),
        pltpu.VMEM((128, 128), jnp.float32),
        pltpu.VMEM((128, 128), jnp.float32),
        pltpu.VMEM((128, 128), jnp.float32),
        pltpu.VMEM((64, 128), jnp.float32),
        pltpu.VMEM((64, 128), jnp.float32),
        pltpu.VMEM((64, 128), jnp.float32),
        pltpu.SemaphoreType.DMA,
        pltpu.SemaphoreType.DMA,
        pltpu.SemaphoreType.DMA,
        pltpu.SemaphoreType.DMA,
        pltpu.SemaphoreType.DMA,
        pltpu.SemaphoreType.DMA,
    ],
)
def _k2(idxT, r_in, out, idx_v, idx2_v, g0, g1, g2, o0, o1, o2,
        gs0, gs1, gs2, os0, os1, os2):
  w = lax.axis_index("s") * NC + lax.axis_index("c")
  iota = lax.iota(jnp.int32, 16)

  pltpu.sync_copy(idxT.at[:, pl.ds(w * 128, 128)], idx_v)

  @plsc.parallel_loop(0, 200, unroll=2)
  def _(s):
    for c0 in range(0, 128, 16):
      v = idx_v[s, pl.ds(c0, 16)]
      idx2_v[s, pl.ds(c0, 16)] = lax.shift_right_logical(v, 1)

  gbufs = (g0, g1, g2)
  gsems = (gs0, gs1, gs2)
  obufs = (o0, o1, o2)
  osems = (os0, os1, os2)

  def start_gather(s, b):
    pltpu.async_copy(r_in.at[idx2_v.at[s]], gbufs[b], gsems[b])

  def wait_gather(b):
    pltpu.make_async_copy(r_in.at[pl.ds(0, 128)], gbufs[b], gsems[b]).wait()

  def transpose_block(s, b):
    g = gbufs[b]
    obuf = obufs[b]
    for b0 in range(0, 128, 16):
      lsb64 = (idx_v[s, pl.ds(b0, 16)] & 1) * 64

      @plsc.parallel_loop(0, 64, unroll=8)
      def _(f):
        v = plsc.load_gather(g, [iota + b0, lsb64 + f])
        obuf[f, pl.ds(b0, 16)] = v

  def start_out(s, b):
    pltpu.async_copy(obufs[b], out.at[s, :, pl.ds(w * 128, 128)], osems[b])

  def wait_out(b):
    pltpu.make_async_copy(
        obufs[b], out.at[0, :, pl.ds(w * 128, 128)], osems[b]).wait()

  start_gather(0, 0)
  start_gather(1, 1)
  start_gather(2, 2)

  @pl.loop(0, 201, step=3)
  def _(s):
    for i in range(3):
      kk = s + i

      @pl.when(kk < 200)
      def _():
        wait_gather(i)

        @pl.when(kk >= 3)
        def _():
          wait_out(i)

        transpose_block(kk, i)
        start_out(kk, i)

        @pl.when(kk + 3 < 200)
        def _():
          start_gather(kk + 3, i)

  wait_out(0)
  wait_out(1)
  wait_out(2)


def kernel(word_indices, table):
  r = _k1(table.T)
  outT = _k2(word_indices.T, r)
  return outT.transpose(2, 0, 1)


# speedup vs baseline: 1.8065x; 1.8065x over previous
import functools

import jax
import jax.numpy as jnp
from jax import lax
from jax.experimental import pallas as pl
from jax.experimental.pallas import tpu as pltpu
from jax.experimental.pallas import tpu_sc as plsc

VOC = 1000000
NB_FULL = 7812
NSB = 2604
SBW = 82
R_ROWS = 500000

NC, NS = 2, 16
NW = NC * NS

_mesh = plsc.VectorSubcoreMesh(
    core_axis_name="c", subcore_axis_name="s", num_cores=NC, num_subcores=NS)

_params = pltpu.CompilerParams(
    use_tc_tiling_on_sc=True, needs_layout_passes=False)


def _splat(x):
  return jnp.zeros((16,), jnp.int32) + x


@functools.partial(
    pl.kernel,
    out_type=jax.ShapeDtypeStruct((R_ROWS, 128), jnp.float32),
    mesh=_mesh,
    compiler_params=_params,
    scratch_types=[
        pltpu.VMEM((2, 64, 384), jnp.float32),
        pltpu.VMEM((2, 192, 128), jnp.float32),
        pltpu.VMEM((64, 64), jnp.float32),
        pltpu.VMEM((32, 128), jnp.float32),
        pltpu.SemaphoreType.DMA,
        pltpu.SemaphoreType.DMA,
        pltpu.SemaphoreType.DMA,
        pltpu.SemaphoreType.DMA,
    ],
)
def _k1(tableT, r_out, in_v, out_v, tin_v, tout_v, is0, is1, os0, os1):
  w = lax.axis_index("s") * NC + lax.axis_index("c")
  iota = lax.iota(jnp.int32, 16)
  isems = (is0, is1)
  osems = (os0, os1)
  j0 = w * SBW

  def start_in(j, b):
    pltpu.async_copy(tableT.at[:, pl.ds(j * 384, 384)], in_v.at[b], isems[b])

  def wait_in(b):
    pltpu.make_async_copy(
        tableT.at[:, pl.ds(0, 384)], in_v.at[b], isems[b]).wait()

  def start_out(j, b):
    pltpu.async_copy(out_v.at[b], r_out.at[pl.ds(j * 192, 192)], osems[b])

  def wait_out(b):
    pltpu.make_async_copy(
        out_v.at[b], r_out.at[pl.ds(0, 192)], osems[b]).wait()

  def transpose_rows(src, dst, nlanes):
    for f0 in range(0, 64, 16):
      fvec = iota + f0

      @plsc.parallel_loop(0, nlanes, step=16)
      def _(l0):
        for d in range(16):
          perm = (iota + d) & 15
          l = l0 + perm
          v = plsc.load_gather(src, [fvec, l])
          plsc.store_scatter(
              dst, [lax.shift_right_logical(l, 1),
                    lax.shift_left((l & 1), 6) + fvec], v)

  @pl.when(j0 < NSB)
  def _():
    start_in(j0, 0)

  @pl.when(j0 + 1 < NSB)
  def _():
    start_in(j0 + 1, 1)

  @pl.loop(0, SBW + 1, step=2)
  def _(k):
    for half in range(2):
      kk = k + half
      j = j0 + kk
      b = half

      @pl.when((j < NSB) & (kk < SBW))
      def _():
        wait_in(b)

        @pl.when(kk >= 2)
        def _():
          wait_out(b)

        transpose_rows(in_v.at[b], out_v.at[b], 384)
        start_out(j, b)

        @pl.when((j + 2 < NSB) & (kk + 2 < SBW))
        def _():
          start_in(j + 2, b)

  @pl.when(j0 < NSB)
  def _():
    wait_out(0)

  @pl.when(j0 + 1 < NSB)
  def _():
    wait_out(1)

  @pl.when(w == 0)
  def _():
    pltpu.sync_copy(tableT.at[:, pl.ds(NB_FULL * 128, 64)], tin_v)

    for f0 in range(0, 64, 16):
      fvec = iota + f0

      @plsc.parallel_loop(0, 64, step=16)
      def _(l0):
        for d in range(16):
          perm = (iota + d) & 15
          l = l0 + perm
          v = plsc.load_gather(tin_v, [fvec, l])
          plsc.store_scatter(
              tout_v, [lax.shift_right_logical(l, 1),
                       lax.shift_left((l & 1), 6) + fvec], v)

    pltpu.sync_copy(tout_v, r_out.at[pl.ds(NB_FULL * 64, 32)])


@functools.partial(
    pl.kernel,
    out_type=jax.ShapeDtypeStruct((200, 64, 4096), jnp.float32),
    mesh=_mesh,
    compiler_params=_params,
    scratch_types=[
        pltpu.VMEM((200, 128), jnp.int32),
        pltpu.VMEM((200, 128), jnp.int32),
        pltpu.VMEM((128, 128), jnp.float32),
        pltpu.VMEM((128, 128), jnp.float32),
        pltpu.VMEM((64, 128), jnp.float32),
        pltpu.VMEM((64, 128), jnp.float32),
        pltpu.SemaphoreType.DMA,
        pltpu.SemaphoreType.DMA,
        pltpu.SemaphoreType.DMA,
        pltpu.SemaphoreType.DMA,
    ],
)
def _k2(idxT, r_in, out, idx_v, idx2_v, g0, g1, o0, o1,
        gs0, gs1, os0, os1):
  w = lax.axis_index("s") * NC + lax.axis_index("c")
  iota = lax.iota(jnp.int32, 16)

  pltpu.sync_copy(idxT.at[:, pl.ds(w * 128, 128)], idx_v)

  @plsc.parallel_loop(0, 200, unroll=2)
  def _(s):
    for c0 in range(0, 128, 16):
      v = idx_v[s, pl.ds(c0, 16)]
      idx2_v[s, pl.ds(c0, 16)] = lax.shift_right_logical(v, 1)

  gbufs = (g0, g1)
  gsems = (gs0, gs1)
  obufs = (o0, o1)
  osems = (os0, os1)

  def start_gather(s, b):
    pltpu.async_copy(r_in.at[idx2_v.at[s]], gbufs[b], gsems[b])

  def wait_gather(b):
    pltpu.make_async_copy(r_in.at[pl.ds(0, 128)], gbufs[b], gsems[b]).wait()

  def transpose_block(s, b):
    g = gbufs[b]
    obuf = obufs[b]
    for b0 in range(0, 128, 16):
      blvec = iota + b0
      lsb64 = (idx_v[s, pl.ds(b0, 16)] & 1) * 64

      @plsc.parallel_loop(0, 64, step=16)
      def _(f0):
        for d in range(16):
          fvec = f0 + ((iota + d) & 15)
          v = plsc.load_gather(g, [blvec, lsb64 + fvec])
          plsc.store_scatter(obuf, [fvec, blvec], v)

  def start_out(s, b):
    pltpu.async_copy(obufs[b], out.at[s, :, pl.ds(w * 128, 128)], osems[b])

  def wait_out(b):
    pltpu.make_async_copy(
        obufs[b], out.at[0, :, pl.ds(w * 128, 128)], osems[b]).wait()

  start_gather(0, 0)
  start_gather(1, 1)

  @pl.loop(0, 200, step=2)
  def _(s):
    for i in range(2):
      kk = s + i

      @pl.when(kk < 200)
      def _():
        wait_gather(i)

        @pl.when(kk >= 2)
        def _():
          wait_out(i)

        transpose_block(kk, i)
        start_out(kk, i)

        @pl.when(kk + 2 < 200)
        def _():
          start_gather(kk + 2, i)

  wait_out(0)
  wait_out(1)


def kernel(word_indices, table):
  r = _k1(table.T)
  outT = _k2(word_indices.T, r)
  return outT.transpose(2, 0, 1)
